# baseline (speedup 1.0000x reference)
import jax
import jax.numpy as jnp
from jax.experimental import pallas as pl
from jax.experimental.pallas import tpu as pltpu

N = 1024
IN_F = 256
OUT_F = 64
ALPHA = 0.2
BR = 256
K = N // BR


def _gat_body(x_ref, adj_ref, w_ref, ap_ref, bias_ref, wres_ref,
              gamma_ref, beta_ref, out_ref,
              haug_s, f_s, g_s, c_s, gm_s, a_s, res_s):
    i = pl.program_id(0)

    @pl.when(i == 0)
    def _prologue():
        x = x_ref[...]
        h = jnp.dot(x, w_ref[...], preferred_element_type=jnp.float32) \
            + bias_ref[...].reshape(1, OUT_F)
        haug_s[...] = jnp.concatenate(
            [h, jnp.ones((N, 1), jnp.float32)], axis=1)
        f = jnp.sum(h * ap_ref[:, :OUT_F], axis=1, keepdims=True)
        g = jnp.sum(h * ap_ref[:, OUT_F:], axis=1, keepdims=True)
        f_s[...] = f
        g_s[...] = g.T
        t = jnp.max(f, axis=0, keepdims=True) \
            + jnp.max(g, axis=0, keepdims=True)
        c_s[...] = jnp.maximum(t, ALPHA * t)
        gm_s[...] = jnp.zeros((1, 1), jnp.float32)
        res_s[...] = jax.lax.dot_general(x, wres_ref[...],
                                         (((1,), (1,)), ((), ())),
                                         preferred_element_type=jnp.float32)

    s = f_s[pl.ds(i * BR, BR), :] + g_s[...]
    s = jnp.maximum(s, ALPHA * s)
    e = jnp.where(adj_ref[...] != 0, jnp.exp(s - c_s[...]), 0.0)
    a_s[pl.ds(i * BR, BR), :] = jnp.dot(
        e, haug_s[...], preferred_element_type=jnp.float32)
    bmax = jnp.max(jnp.max(e, axis=1, keepdims=True), axis=0,
                   keepdims=True)
    gm_s[...] = jnp.maximum(gm_s[...], bmax)

    @pl.when(i == K - 1)
    def _epilogue():
        corr = 1e-8 * gm_s[...]
        aaug = a_s[...]
        hp = aaug[:, :OUT_F] / (aaug[:, OUT_F:] + corr) + res_s[...]
        mean = jnp.mean(hp, axis=-1, keepdims=True)
        cen = hp - mean
        var = jnp.mean(cen * cen, axis=-1, keepdims=True)
        hn = cen * jax.lax.rsqrt(var + 1e-5) \
            * gamma_ref[...].reshape(1, OUT_F) \
            + beta_ref[...].reshape(1, OUT_F)
        out_ref[...] = jnp.where(hn > 0, hn,
                                 jnp.exp(jnp.minimum(hn, 0.0)) - 1.0)


def kernel(input, adj, W, a_param, bias, W_res, ln_gamma, ln_beta):
    full = lambda r, c: pl.BlockSpec((r, c), lambda i: (0, 0))
    return pl.pallas_call(
        _gat_body,
        grid=(K,),
        in_specs=[
            full(N, IN_F),
            pl.BlockSpec((BR, N), lambda i: (i, 0)),
            full(IN_F, OUT_F),
            full(1, 2 * OUT_F),
            pl.BlockSpec((OUT_F,), lambda i: (0,)),
            full(OUT_F, IN_F),
            pl.BlockSpec((OUT_F,), lambda i: (0,)),
            pl.BlockSpec((OUT_F,), lambda i: (0,)),
        ],
        out_specs=full(N, OUT_F),
        out_shape=jax.ShapeDtypeStruct((N, OUT_F), jnp.float32),
        scratch_shapes=[
            pltpu.VMEM((N, OUT_F + 1), jnp.float32),
            pltpu.VMEM((N, 1), jnp.float32),
            pltpu.VMEM((1, N), jnp.float32),
            pltpu.VMEM((1, 1), jnp.float32),
            pltpu.VMEM((1, 1), jnp.float32),
            pltpu.VMEM((N, OUT_F + 1), jnp.float32),
            pltpu.VMEM((N, OUT_F), jnp.float32),
        ],
        compiler_params=pltpu.CompilerParams(
            dimension_semantics=("arbitrary",)),
    )(input, adj, W, a_param, bias, W_res, ln_gamma, ln_beta)
